# baseline (speedup 1.0000x reference)
import functools

import jax
import jax.numpy as jnp
from jax import lax
from jax.experimental import pallas as pl
from jax.experimental.pallas import tpu as pltpu
from jax.experimental.pallas import tpu_sc as plsc

_NEG_BIG = -1e30


def _gate_tc_body(x_ref, wcat_ref, b1c_ref, w2c_ref, gate_ref, y_ref, *, bn, n_valid, row0):
    i = pl.program_id(0)
    xb = x_ref[...]
    hy = jnp.dot(xb, wcat_ref[...], preferred_element_type=jnp.float32)
    hyT = hy.T
    h = jnp.maximum(hyT + b1c_ref[...], 0.0)
    gate = jnp.sum(h * w2c_ref[...], axis=0, keepdims=True)
    y = hyT[16:17, :]
    cols = row0 + i * bn + lax.broadcasted_iota(jnp.int32, (1, bn), 1)
    valid = cols < n_valid
    gate = jnp.where(valid, gate, _NEG_BIG)
    gate_ref[...] = jnp.reshape(gate, (bn,))
    y_ref[...] = jnp.reshape(jnp.where(valid, y, 0.0), (bn,))


def _seg_sc_body(gate_hbm, y_hbm, ids_hbm, den_hbm, num_hbm,
                 gate_v, y_v, ids_v, den_v, num_v,
                 sem1, sem2, sem3,
                 *, ch, nsub, g, unroll):
    nv = ch // 16
    c = lax.axis_index("c")
    s = lax.axis_index("s")
    wid = c * nsub + s
    base = wid * ch

    h1 = pltpu.async_copy(gate_hbm.at[pl.ds(base, ch)], gate_v, sem1)
    h2 = pltpu.async_copy(y_hbm.at[pl.ds(base, ch)], y_v, sem2)
    h3 = pltpu.async_copy(ids_hbm.at[pl.ds(base, ch)], ids_v, sem3)

    zz = jnp.zeros((16,), jnp.float32)

    def z_body(k, carry):
        den_v[pl.ds(k * 16, 16)] = zz
        num_v[pl.ds(k * 16, 16)] = zz
        return carry

    lax.fori_loop(0, g // 16, z_body, 0)
    h1.wait()
    h2.wait()
    h3.wait()

    def acc_body(j, carry):
        for u in range(unroll):
            off = (j * unroll + u) * 16
            gv = gate_v[pl.ds(off, 16)]
            yv = y_v[pl.ds(off, 16)]
            iv = ids_v[pl.ds(off, 16)]
            e = jnp.exp(gv)
            plsc.addupdate_scatter(den_v, [iv], e)
            plsc.addupdate_scatter(num_v, [iv], e * yv)
        return carry

    lax.fori_loop(0, nv // unroll, acc_body, 0)

    pltpu.sync_copy(den_v, den_hbm.at[wid])
    pltpu.sync_copy(num_v, num_hbm.at[wid])


def _fin_tc_body(da_ref, db_ref, na_ref, nb_ref, bb_ref, out_ref):
    den = (jnp.sum(da_ref[...], axis=0, keepdims=True)
           + jnp.sum(db_ref[...], axis=0, keepdims=True))
    num = (jnp.sum(na_ref[...], axis=0, keepdims=True)
           + jnp.sum(nb_ref[...], axis=0, keepdims=True))
    out_ref[...] = num / (den + 1e-16) + bb_ref[...]


def kernel(x, batch, W1, b1, W2, b2, Wout, bout):
    n, cdim = x.shape
    hdim = W1.shape[1]
    g = 512
    nsub = 16
    nw = 2 * nsub
    bn = 5120
    nb = pl.cdiv(n, bn)
    n_pad = nb * bn
    assert n_pad % (nw * 16) == 0
    ch = n_pad // nw

    f32 = jnp.float32

    wcat = jnp.zeros((cdim, 32), f32)
    wcat = wcat.at[:, :hdim].set(W1)
    wcat = wcat.at[:, 16].set(Wout[:, 0])
    b1c = jnp.zeros((32, 1), f32).at[:hdim, 0].set(b1)
    w2c = jnp.zeros((32, 1), f32).at[:hdim, 0].set(W2[:, 0])

    ids = jnp.pad(batch.astype(jnp.int32), (0, n_pad - n), constant_values=g - 1)

    n_half = n_pad // 2
    nbh = nb // 2
    ch = n_half // nw
    assert ch % 16 == 0

    mesh = plsc.VectorSubcoreMesh(core_axis_name="c", subcore_axis_name="s")
    parts = []
    for half in range(2):
        off = half * nbh
        gate1d, y1d = pl.pallas_call(
            functools.partial(_gate_tc_body, bn=bn, n_valid=n, row0=off * bn),
            grid=(nbh,),
            in_specs=[
                pl.BlockSpec((bn, cdim), lambda i, off=off: (i + off, 0)),
                pl.BlockSpec((cdim, 32), lambda i: (0, 0)),
                pl.BlockSpec((32, 1), lambda i: (0, 0)),
                pl.BlockSpec((32, 1), lambda i: (0, 0)),
            ],
            out_specs=[
                pl.BlockSpec((bn,), lambda i: (i,)),
                pl.BlockSpec((bn,), lambda i: (i,)),
            ],
            out_shape=[
                jax.ShapeDtypeStruct((n_half,), f32),
                jax.ShapeDtypeStruct((n_half,), f32),
            ],
        )(x, wcat, b1c, w2c)

        ids_h = lax.slice(ids, (half * n_half,), ((half + 1) * n_half,))
        sc_fn = functools.partial(
            pl.kernel,
            mesh=mesh,
            compiler_params=pltpu.CompilerParams(needs_layout_passes=False),
            out_type=(
                jax.ShapeDtypeStruct((nw, g), f32),
                jax.ShapeDtypeStruct((nw, g), f32),
            ),
            scratch_types=[
                pltpu.VMEM((ch,), f32),
                pltpu.VMEM((ch,), f32),
                pltpu.VMEM((ch,), jnp.int32),
                pltpu.VMEM((g,), f32),
                pltpu.VMEM((g,), f32),
                pltpu.SemaphoreType.DMA,
                pltpu.SemaphoreType.DMA,
                pltpu.SemaphoreType.DMA,
            ],
        )(functools.partial(_seg_sc_body, ch=ch, nsub=nsub, g=g, unroll=10))
        parts.append(sc_fn(gate1d, y1d, ids_h))

    (den_a, num_a), (den_b, num_b) = parts
    bb = bout.astype(f32).reshape(1, 1)
    out = pl.pallas_call(
        _fin_tc_body,
        out_shape=jax.ShapeDtypeStruct((1, g), f32),
    )(den_a, den_b, num_a, num_b, bb)
    return out.reshape(g, 1)


# speedup vs baseline: 1.0518x; 1.0518x over previous
import functools

import jax
import jax.numpy as jnp
from jax import lax
from jax.experimental import pallas as pl
from jax.experimental.pallas import tpu as pltpu
from jax.experimental.pallas import tpu_sc as plsc

_NEG_BIG = -1e30


def _gate_tc_body(x_ref, wcat_ref, b1c_ref, w2c_ref, gate_ref, y_ref, *, bn, n_valid):
    i = pl.program_id(0)
    xb = x_ref[...]
    hy = jnp.dot(xb, wcat_ref[...], preferred_element_type=jnp.float32)
    hyT = hy.T
    h = jnp.maximum(hyT + b1c_ref[...], 0.0)
    gate = jnp.sum(h * w2c_ref[...], axis=0, keepdims=True)
    y = hyT[16:17, :]
    cols = i * bn + lax.broadcasted_iota(jnp.int32, (1, bn), 1)
    valid = cols < n_valid
    gate = jnp.where(valid, gate, _NEG_BIG)
    gate_ref[...] = jnp.reshape(gate, (bn,))
    y_ref[...] = jnp.reshape(jnp.where(valid, y, 0.0), (bn,))


def _seg_sc_body(gate_hbm, y_hbm, ids_hbm, den_hbm, num_hbm,
                 gate_v, y_v, ids_v, den_v, num_v,
                 sem1, sem2, sem3,
                 *, ch, nsub, g, unroll):
    nv = ch // 16
    c = lax.axis_index("c")
    s = lax.axis_index("s")
    wid = c * nsub + s
    base = wid * ch

    h1 = pltpu.async_copy(gate_hbm.at[pl.ds(base, ch)], gate_v, sem1)
    h2 = pltpu.async_copy(y_hbm.at[pl.ds(base, ch)], y_v, sem2)
    h3 = pltpu.async_copy(ids_hbm.at[pl.ds(base, ch)], ids_v, sem3)

    zz = jnp.zeros((16,), jnp.float32)

    def z_body(l, carry):
        def z_inner(k, carry2):
            den_v[l, pl.ds(k * 16, 16)] = zz
            num_v[l, pl.ds(k * 16, 16)] = zz
            return carry2

        return lax.fori_loop(0, g // 16, z_inner, carry)

    lax.fori_loop(0, 16, z_body, 0)
    h1.wait()
    h2.wait()
    h3.wait()

    lane = lax.iota(jnp.int32, 16)

    def acc_body(j, carry):
        for u in range(unroll):
            off = (j * unroll + u) * 16
            gv = gate_v[pl.ds(off, 16)]
            yv = y_v[pl.ds(off, 16)]
            iv = ids_v[pl.ds(off, 16)]
            e = jnp.exp(gv)
            plsc.addupdate_scatter(den_v, [lane, iv], e)
            plsc.addupdate_scatter(num_v, [lane, iv], e * yv)
        return carry

    lax.fori_loop(0, nv // unroll, acc_body, 0)

    pltpu.sync_copy(den_v, den_hbm.at[pl.ds(wid * 16, 16)])
    pltpu.sync_copy(num_v, num_hbm.at[pl.ds(wid * 16, 16)])


def _fin_tc_body(den_ref, num_ref, bb_ref, out_ref, *, g):
    den = jnp.sum(den_ref[...], axis=0, keepdims=True)[:, :g]
    num = jnp.sum(num_ref[...], axis=0, keepdims=True)[:, :g]
    out_ref[...] = num / (den + 1e-16) + bb_ref[...]


def kernel(x, batch, W1, b1, W2, b2, Wout, bout):
    n, cdim = x.shape
    hdim = W1.shape[1]
    g = 512
    nsub = 16
    nw = 2 * nsub
    bn = 7168
    nb = pl.cdiv(n, bn)
    n_pad = nb * bn
    assert n_pad % (nw * 16) == 0
    ch = n_pad // nw

    f32 = jnp.float32

    wcat = jnp.zeros((cdim, 32), f32)
    wcat = wcat.at[:, :hdim].set(W1)
    wcat = wcat.at[:, 16].set(Wout[:, 0])
    b1c = jnp.zeros((32, 1), f32).at[:hdim, 0].set(b1)
    w2c = jnp.zeros((32, 1), f32).at[:hdim, 0].set(W2[:, 0])

    tc_outs = pl.pallas_call(
        functools.partial(_gate_tc_body, bn=bn, n_valid=n),
        grid=(nb,),
        in_specs=[
            pl.BlockSpec((bn, cdim), lambda i: (i, 0)),
            pl.BlockSpec((cdim, 32), lambda i: (0, 0)),
            pl.BlockSpec((32, 1), lambda i: (0, 0)),
            pl.BlockSpec((32, 1), lambda i: (0, 0)),
        ],
        out_specs=[
            pl.BlockSpec((bn,), lambda i: (i,)),
            pl.BlockSpec((bn,), lambda i: (i,)),
        ],
        out_shape=[
            jax.ShapeDtypeStruct((n_pad,), f32),
            jax.ShapeDtypeStruct((n_pad,), f32),
        ],
    )(x, wcat, b1c, w2c)

    gate1d, y1d = tc_outs
    ids = jnp.pad(batch.astype(jnp.int32), (0, n_pad - n), constant_values=g - 1)

    mesh = plsc.VectorSubcoreMesh(core_axis_name="c", subcore_axis_name="s")
    gp = g + 1
    sc_fn = functools.partial(
        pl.kernel,
        mesh=mesh,
        compiler_params=pltpu.CompilerParams(needs_layout_passes=False),
        out_type=(
            jax.ShapeDtypeStruct((nw * 16, gp), f32),
            jax.ShapeDtypeStruct((nw * 16, gp), f32),
        ),
        scratch_types=[
            pltpu.VMEM((ch,), f32),
            pltpu.VMEM((ch,), f32),
            pltpu.VMEM((ch,), jnp.int32),
            pltpu.VMEM((16, gp), f32),
            pltpu.VMEM((16, gp), f32),
            pltpu.SemaphoreType.DMA,
            pltpu.SemaphoreType.DMA,
            pltpu.SemaphoreType.DMA,
        ],
    )(functools.partial(_seg_sc_body, ch=ch, nsub=nsub, g=g, unroll=14))

    den_parts, num_parts = sc_fn(gate1d, y1d, ids)

    bb = bout.astype(f32).reshape(1, 1)
    out = pl.pallas_call(
        functools.partial(_fin_tc_body, g=g),
        out_shape=jax.ShapeDtypeStruct((1, g), f32),
    )(den_parts, num_parts, bb)
    return out.reshape(g, 1)
